# baseline (speedup 1.0000x reference)
import functools

import jax
import jax.numpy as jnp
import numpy as np
from jax import lax
from jax.experimental import pallas as pl
from jax.experimental.pallas import tpu as pltpu
from jax.experimental.pallas import tpu_sc as plsc

N_NODES = 10000
N_EDGES = 160000
NC = 9
CH = 64
NCH = NC * CH
RESCALE = 5.0
M0_COLS = (0, 2, 6)

EDGE_BLK = 2000

N_SC = 2
N_TILES = 16
N_CHUNK = 2
E_CHU = N_EDGES // N_CHUNK
SC_WIN = 1000
EPT = E_CHU // N_TILES
NPT = 632
NPT_LAST = N_NODES - NPT * (N_TILES - 1)


S1_WIN = 200
S1_EPW = N_EDGES // (N_SC * N_TILES)


def _node_table_body(anf_ref, st_ref, tt_ref, w1s_ref, w1t_ref, b1_ref,
                     psrc_ref, ptgt_ref):
    f32 = jnp.float32
    a_s = jnp.dot(st_ref[...], w1s_ref[...], preferred_element_type=f32)
    a_t = jnp.dot(tt_ref[...], w1t_ref[...], preferred_element_type=f32)
    ids = jax.lax.broadcasted_iota(jnp.int32, (1, 90), 1).astype(f32)
    oh = (anf_ref[...] == ids).astype(f32)
    psrc_ref[...] = jnp.dot(oh, a_s, preferred_element_type=f32) + b1_ref[...]
    ptgt_ref[...] = jnp.dot(oh, a_t, preferred_element_type=f32)


@jax.jit
def _node_table_stage(anf, source_table, target_table, W1s, W1t, b1):
    blk = 2000
    return pl.pallas_call(
        _node_table_body,
        grid=(N_NODES // blk,),
        in_specs=[
            pl.BlockSpec((blk, 1), lambda i: (i, 0)),
            pl.BlockSpec((90, 64), lambda i: (0, 0)),
            pl.BlockSpec((90, 64), lambda i: (0, 0)),
            pl.BlockSpec((64, 64), lambda i: (0, 0)),
            pl.BlockSpec((64, 64), lambda i: (0, 0)),
            pl.BlockSpec((1, 64), lambda i: (0, 0)),
        ],
        out_specs=[pl.BlockSpec((blk, 64), lambda i: (i, 0)),
                   pl.BlockSpec((blk, 64), lambda i: (i, 0))],
        out_shape=[jax.ShapeDtypeStruct((N_NODES, 64), jnp.float32),
                   jax.ShapeDtypeStruct((N_NODES, 64), jnp.float32)],
    )(anf, source_table, target_table, W1s, W1t, b1)


def _gather_body(esrc_hbm, etgt_hbm, psrc_hbm, ptgt_hbm, out_hbm,
                 src_v, tgt_v, rows_s, rows_t, psp, ptp, sem):
    c = lax.axis_index("c")
    s = lax.axis_index("s")
    w = s * N_SC + c

    @pl.when(s == 0)
    def _():
        pltpu.sync_copy(psrc_hbm, psp)
        pltpu.sync_copy(ptgt_hbm, ptp)

    plsc.subcore_barrier()
    base = w * S1_EPW
    for k in range(S1_EPW // S1_WIN):
        e0 = base + k * S1_WIN
        pltpu.sync_copy(esrc_hbm.at[pl.ds(e0, S1_WIN)], src_v)
        pltpu.sync_copy(etgt_hbm.at[pl.ds(e0, S1_WIN)], tgt_v)
        pltpu.async_copy(psp.at[src_v], rows_s, sem).wait()
        pltpu.async_copy(ptp.at[tgt_v], rows_t, sem).wait()
        pltpu.sync_copy(rows_s, out_hbm.at[0, pl.ds(e0, S1_WIN)])
        pltpu.sync_copy(rows_t, out_hbm.at[1, pl.ds(e0, S1_WIN)])


@jax.jit
def _gather_stage(esrc, etgt, psrc, ptgt):
    mesh = plsc.VectorSubcoreMesh(core_axis_name="c", subcore_axis_name="s")
    f = functools.partial(
        pl.kernel,
        mesh=mesh,
        out_type=jax.ShapeDtypeStruct((2, N_EDGES, 64), jnp.float32),
        scratch_types=[
            pltpu.VMEM((S1_WIN,), jnp.int32),
            pltpu.VMEM((S1_WIN,), jnp.int32),
            pltpu.VMEM((S1_WIN, 64), jnp.float32),
            pltpu.VMEM((S1_WIN, 64), jnp.float32),
            pltpu.VMEM_SHARED((N_NODES, 64), jnp.float32),
            pltpu.VMEM_SHARED((N_NODES, 64), jnp.float32),
            pltpu.SemaphoreType.DMA,
        ],
        compiler_params=pltpu.CompilerParams(use_tc_tiling_on_sc=False),
    )(_gather_body)
    return f(esrc, etgt, psrc, ptgt)


def _mlp_wigner_body(dist_ref, g_ref, wig_ref,
                     w1d_ref, g1_ref, be1_ref,
                     w2_ref, b2_ref, g2_ref, be2_ref,
                     w3_ref, b3_ref, out_ref):
    f32 = jnp.float32
    h = (jnp.dot(dist_ref[...], w1d_ref[...], preferred_element_type=f32)
         + g_ref[0] + g_ref[1])
    mu = jnp.mean(h, axis=-1, keepdims=True)
    var = jnp.mean((h - mu) ** 2, axis=-1, keepdims=True)
    h = (h - mu) * jax.lax.rsqrt(var + 1e-5) * g1_ref[...] + be1_ref[...]
    h = h * jax.nn.sigmoid(h)
    h = jnp.dot(h, w2_ref[...], preferred_element_type=f32) + b2_ref[...]
    mu = jnp.mean(h, axis=-1, keepdims=True)
    var = jnp.mean((h - mu) ** 2, axis=-1, keepdims=True)
    h = (h - mu) * jax.lax.rsqrt(var + 1e-5) * g2_ref[...] + be2_ref[...]
    h = h * jax.nn.sigmoid(h)
    xm0 = jnp.dot(h, w3_ref[...], preferred_element_type=f32) + b3_ref[...]
    x0 = xm0[:, 0:CH]
    x1 = xm0[:, CH:2 * CH]
    x2 = xm0[:, 2 * CH:3 * CH]
    for i in range(NC):
        w0 = wig_ref[:, 3 * i + 0][:, None]
        w1 = wig_ref[:, 3 * i + 1][:, None]
        w2 = wig_ref[:, 3 * i + 2][:, None]
        out_ref[i] = w0 * x0 + w1 * x1 + w2 * x2


@jax.jit
def _edge_stage(edge_distance, g, wig3,
                W1, g1, beta1, W2, b2, g2, beta2, W3, b3):
    w3s = W3 / RESCALE
    b3s = b3 / RESCALE
    blk = EDGE_BLK
    ne = edge_distance.shape[0]
    nblk = ne // blk
    row = lambda i: (i, 0)
    zero = lambda i: (0, 0)
    return pl.pallas_call(
        _mlp_wigner_body,
        grid=(nblk,),
        in_specs=[
            pl.BlockSpec((blk, 128), row),
            pl.BlockSpec((2, blk, 64), lambda i: (0, i, 0)),
            pl.BlockSpec((blk, 27), row),
            pl.BlockSpec((128, 64), zero),
            pl.BlockSpec((1, 64), zero),
            pl.BlockSpec((1, 64), zero),
            pl.BlockSpec((64, 64), zero),
            pl.BlockSpec((1, 64), zero),
            pl.BlockSpec((1, 64), zero),
            pl.BlockSpec((1, 64), zero),
            pl.BlockSpec((64, 192), zero),
            pl.BlockSpec((1, 192), zero),
        ],
        out_specs=pl.BlockSpec((NC, blk, CH), lambda i: (0, i, 0)),
        out_shape=jax.ShapeDtypeStruct((NC, ne, CH), jnp.float32),
    )(edge_distance, g, wig3,
      W1[:128], g1[None], beta1[None],
      W2, b2[None], g2[None], beta2[None], w3s, b3s[None])


def _scatter_body(xrot_hbm, tgt_hbm, zeros_hbm, out_hbm, idx_v, wbuf, acc):
    c = lax.axis_index("c")
    s = lax.axis_index("s")
    for pi in range((NC + N_SC - 1) // N_SC):
        coeff = c + N_SC * pi

        @pl.when(coeff < NC)
        def _():
            @pl.when(s < N_TILES - 1)
            def _():
                pltpu.sync_copy(zeros_hbm, acc.at[pl.ds(s * NPT, NPT)])

            @pl.when(s == N_TILES - 1)
            def _():
                pltpu.sync_copy(zeros_hbm.at[pl.ds(0, NPT_LAST)],
                                acc.at[pl.ds((N_TILES - 1) * NPT, NPT_LAST)])

            plsc.subcore_barrier()

            def win(w, carry):
                e0 = s * EPT + w * SC_WIN
                pltpu.sync_copy(tgt_hbm.at[pl.ds(e0, SC_WIN)], idx_v)
                pltpu.sync_copy(xrot_hbm.at[coeff, pl.ds(e0, SC_WIN)], wbuf)
                pltpu.sync_copy(wbuf, acc.at[idx_v], add=True)
                return carry

            lax.fori_loop(0, EPT // SC_WIN, win, 0)
            plsc.subcore_barrier()

            @pl.when(s < N_TILES - 1)
            def _():
                pltpu.sync_copy(acc.at[pl.ds(s * NPT, NPT)],
                                out_hbm.at[coeff, pl.ds(s * NPT, NPT)])

            @pl.when(s == N_TILES - 1)
            def _():
                pltpu.sync_copy(acc.at[pl.ds((N_TILES - 1) * NPT, NPT_LAST)],
                                out_hbm.at[coeff,
                                           pl.ds((N_TILES - 1) * NPT, NPT_LAST)])

            plsc.subcore_barrier()


@jax.jit
def _scatter_stage(xrot, tgt, zeros):
    mesh = plsc.VectorSubcoreMesh(core_axis_name="c", subcore_axis_name="s")
    f = functools.partial(
        pl.kernel,
        mesh=mesh,
        out_type=jax.ShapeDtypeStruct((NC, N_NODES, CH), jnp.float32),
        scratch_types=[
            pltpu.VMEM((SC_WIN,), jnp.int32),
            pltpu.VMEM((SC_WIN, CH), jnp.float32),
            pltpu.VMEM_SHARED((N_NODES, CH), jnp.float32),
        ],
        compiler_params=pltpu.CompilerParams(use_tc_tiling_on_sc=False),
    )(_scatter_body)
    return f(xrot, tgt, zeros)


def _add_body(a_ref, b_ref, o_ref):
    o_ref[...] = a_ref[...] + b_ref[...]


@jax.jit
def _add_stage(a, b):
    blk = 2000
    spec = pl.BlockSpec((1, blk, CH), lambda i, j: (i, j, 0))
    return pl.pallas_call(
        _add_body,
        grid=(NC, N_NODES // blk),
        in_specs=[spec, spec],
        out_specs=spec,
        out_shape=jax.ShapeDtypeStruct((NC, N_NODES, CH), jnp.float32),
    )(a, b)


def kernel(atomic_numbers, edge_distance, edge_index, source_table, target_table,
           W1, b1, g1, beta1, W2, b2, g2, beta2, W3, b3, wigner_inv):
    anf = atomic_numbers.astype(jnp.float32).reshape(N_NODES, 1)
    psrc, ptgt = _node_table_stage(anf, source_table, target_table,
                                   W1[128:192], W1[192:256], b1[None])
    g = _gather_stage(edge_index[0], edge_index[1], psrc, ptgt)
    wig3 = wigner_inv[:, :, M0_COLS].reshape(N_EDGES, NC * 3)
    zeros = jnp.zeros((NPT, CH), jnp.float32)
    parts = []
    for h in range(N_CHUNK):
        sl = slice(h * E_CHU, (h + 1) * E_CHU)
        x_h = _edge_stage(edge_distance[sl], g[:, sl], wig3[sl],
                          W1, g1, beta1, W2, b2, g2, beta2, W3, b3)
        parts.append(_scatter_stage(x_h, edge_index[1, sl], zeros))
    out = _add_stage(parts[0], parts[1])
    return out.transpose(1, 0, 2)


# speedup vs baseline: 1.0358x; 1.0358x over previous
import functools

import jax
import jax.numpy as jnp
import numpy as np
from jax import lax
from jax.experimental import pallas as pl
from jax.experimental.pallas import tpu as pltpu
from jax.experimental.pallas import tpu_sc as plsc

N_NODES = 10000
N_EDGES = 160000
NC = 9
CH = 64
NCH = NC * CH
RESCALE = 5.0
M0_COLS = (0, 2, 6)

EDGE_BLK = 2000

N_SC = 2
N_TILES = 16
PASSES = 8
PCH = NCH // PASSES
SC_WIN = 1000
EPT = N_EDGES // N_TILES
NPT = 632
NPT_LAST = N_NODES - NPT * (N_TILES - 1)


S1_WIN = 200
S1_EPW = N_EDGES // (N_SC * N_TILES)


def _node_table_body(anf_ref, st_ref, tt_ref, w1s_ref, w1t_ref, b1_ref,
                     psrc_ref, ptgt_ref):
    f32 = jnp.float32
    a_s = jnp.dot(st_ref[...], w1s_ref[...], preferred_element_type=f32)
    a_t = jnp.dot(tt_ref[...], w1t_ref[...], preferred_element_type=f32)
    ids = jax.lax.broadcasted_iota(jnp.int32, (1, 90), 1).astype(f32)
    oh = (anf_ref[...] == ids).astype(f32)
    psrc_ref[...] = jnp.dot(oh, a_s, preferred_element_type=f32) + b1_ref[...]
    ptgt_ref[...] = jnp.dot(oh, a_t, preferred_element_type=f32)


@jax.jit
def _node_table_stage(anf, source_table, target_table, W1s, W1t, b1):
    blk = 2000
    return pl.pallas_call(
        _node_table_body,
        grid=(N_NODES // blk,),
        in_specs=[
            pl.BlockSpec((blk, 1), lambda i: (i, 0)),
            pl.BlockSpec((90, 64), lambda i: (0, 0)),
            pl.BlockSpec((90, 64), lambda i: (0, 0)),
            pl.BlockSpec((64, 64), lambda i: (0, 0)),
            pl.BlockSpec((64, 64), lambda i: (0, 0)),
            pl.BlockSpec((1, 64), lambda i: (0, 0)),
        ],
        out_specs=[pl.BlockSpec((blk, 64), lambda i: (i, 0)),
                   pl.BlockSpec((blk, 64), lambda i: (i, 0))],
        out_shape=[jax.ShapeDtypeStruct((N_NODES, 64), jnp.float32),
                   jax.ShapeDtypeStruct((N_NODES, 64), jnp.float32)],
    )(anf, source_table, target_table, W1s, W1t, b1)


def _gather_body(esrc_hbm, etgt_hbm, psrc_hbm, ptgt_hbm, out_hbm,
                 src_v, tgt_v, rows_s, rows_t, psp, ptp, sem):
    c = lax.axis_index("c")
    s = lax.axis_index("s")
    w = s * N_SC + c

    @pl.when(s == 0)
    def _():
        pltpu.sync_copy(psrc_hbm, psp)
        pltpu.sync_copy(ptgt_hbm, ptp)

    plsc.subcore_barrier()
    base = w * S1_EPW
    for k in range(S1_EPW // S1_WIN):
        e0 = base + k * S1_WIN
        pltpu.sync_copy(esrc_hbm.at[pl.ds(e0, S1_WIN)], src_v)
        pltpu.sync_copy(etgt_hbm.at[pl.ds(e0, S1_WIN)], tgt_v)
        pltpu.async_copy(psp.at[src_v], rows_s, sem).wait()
        pltpu.async_copy(ptp.at[tgt_v], rows_t, sem).wait()
        pltpu.sync_copy(rows_s, out_hbm.at[0, pl.ds(e0, S1_WIN)])
        pltpu.sync_copy(rows_t, out_hbm.at[1, pl.ds(e0, S1_WIN)])


@jax.jit
def _gather_stage(esrc, etgt, psrc, ptgt):
    mesh = plsc.VectorSubcoreMesh(core_axis_name="c", subcore_axis_name="s")
    f = functools.partial(
        pl.kernel,
        mesh=mesh,
        out_type=jax.ShapeDtypeStruct((2, N_EDGES, 64), jnp.float32),
        scratch_types=[
            pltpu.VMEM((S1_WIN,), jnp.int32),
            pltpu.VMEM((S1_WIN,), jnp.int32),
            pltpu.VMEM((S1_WIN, 64), jnp.float32),
            pltpu.VMEM((S1_WIN, 64), jnp.float32),
            pltpu.VMEM_SHARED((N_NODES, 64), jnp.float32),
            pltpu.VMEM_SHARED((N_NODES, 64), jnp.float32),
            pltpu.SemaphoreType.DMA,
        ],
        compiler_params=pltpu.CompilerParams(use_tc_tiling_on_sc=False),
    )(_gather_body)
    return f(esrc, etgt, psrc, ptgt)


def _mlp_wigner_body(dist_ref, g_ref, wig_ref,
                     w1d_ref, g1_ref, be1_ref,
                     w2_ref, b2_ref, g2_ref, be2_ref,
                     w3_ref, b3_ref, out_ref):
    f32 = jnp.float32
    h = (jnp.dot(dist_ref[...], w1d_ref[...], preferred_element_type=f32)
         + g_ref[0] + g_ref[1])
    mu = jnp.mean(h, axis=-1, keepdims=True)
    var = jnp.mean((h - mu) ** 2, axis=-1, keepdims=True)
    h = (h - mu) * jax.lax.rsqrt(var + 1e-5) * g1_ref[...] + be1_ref[...]
    h = h * jax.nn.sigmoid(h)
    h = jnp.dot(h, w2_ref[...], preferred_element_type=f32) + b2_ref[...]
    mu = jnp.mean(h, axis=-1, keepdims=True)
    var = jnp.mean((h - mu) ** 2, axis=-1, keepdims=True)
    h = (h - mu) * jax.lax.rsqrt(var + 1e-5) * g2_ref[...] + be2_ref[...]
    h = h * jax.nn.sigmoid(h)
    xm0 = jnp.dot(h, w3_ref[...], preferred_element_type=f32) + b3_ref[...]
    x0 = xm0[:, 0:CH]
    x1 = xm0[:, CH:2 * CH]
    x2 = xm0[:, 2 * CH:3 * CH]
    for i in range(NC):
        w0 = wig_ref[:, 3 * i + 0][:, None]
        w1 = wig_ref[:, 3 * i + 1][:, None]
        w2 = wig_ref[:, 3 * i + 2][:, None]
        out_ref[i] = w0 * x0 + w1 * x1 + w2 * x2


@jax.jit
def _edge_stage(edge_distance, g, wig3,
                W1, g1, beta1, W2, b2, g2, beta2, W3, b3):
    w3s = W3 / RESCALE
    b3s = b3 / RESCALE
    blk = EDGE_BLK
    nblk = N_EDGES // blk
    row = lambda i: (i, 0)
    zero = lambda i: (0, 0)
    return pl.pallas_call(
        _mlp_wigner_body,
        grid=(nblk,),
        in_specs=[
            pl.BlockSpec((blk, 128), row),
            pl.BlockSpec((2, blk, 64), lambda i: (0, i, 0)),
            pl.BlockSpec((blk, 27), row),
            pl.BlockSpec((128, 64), zero),
            pl.BlockSpec((1, 64), zero),
            pl.BlockSpec((1, 64), zero),
            pl.BlockSpec((64, 64), zero),
            pl.BlockSpec((1, 64), zero),
            pl.BlockSpec((1, 64), zero),
            pl.BlockSpec((1, 64), zero),
            pl.BlockSpec((64, 192), zero),
            pl.BlockSpec((1, 192), zero),
        ],
        out_specs=pl.BlockSpec((NC, blk, CH), lambda i: (0, i, 0)),
        out_shape=jax.ShapeDtypeStruct((NC, N_EDGES, CH), jnp.float32),
    )(edge_distance, g, wig3,
      W1[:128], g1[None], beta1[None],
      W2, b2[None], g2[None], beta2[None], w3s, b3s[None])


def _scatter_body(xrot_hbm, tgt_hbm, zeros_hbm, out_hbm, idx_v, wbuf, acc):
    c = lax.axis_index("c")
    s = lax.axis_index("s")
    for pi in range((NC + N_SC - 1) // N_SC):
        coeff = c + N_SC * pi

        @pl.when(coeff < NC)
        def _():
            @pl.when(s < N_TILES - 1)
            def _():
                pltpu.sync_copy(zeros_hbm, acc.at[pl.ds(s * NPT, NPT)])

            @pl.when(s == N_TILES - 1)
            def _():
                pltpu.sync_copy(zeros_hbm.at[pl.ds(0, NPT_LAST)],
                                acc.at[pl.ds((N_TILES - 1) * NPT, NPT_LAST)])

            plsc.subcore_barrier()

            def win(w, carry):
                e0 = s * EPT + w * SC_WIN
                pltpu.sync_copy(tgt_hbm.at[pl.ds(e0, SC_WIN)], idx_v)
                pltpu.sync_copy(xrot_hbm.at[coeff, pl.ds(e0, SC_WIN)], wbuf)
                pltpu.sync_copy(wbuf, acc.at[idx_v], add=True)
                return carry

            lax.fori_loop(0, EPT // SC_WIN, win, 0)
            plsc.subcore_barrier()

            @pl.when(s < N_TILES - 1)
            def _():
                pltpu.sync_copy(acc.at[pl.ds(s * NPT, NPT)],
                                out_hbm.at[coeff, pl.ds(s * NPT, NPT)])

            @pl.when(s == N_TILES - 1)
            def _():
                pltpu.sync_copy(acc.at[pl.ds((N_TILES - 1) * NPT, NPT_LAST)],
                                out_hbm.at[coeff,
                                           pl.ds((N_TILES - 1) * NPT, NPT_LAST)])

            plsc.subcore_barrier()


@jax.jit
def _scatter_stage(xrot, tgt, zeros):
    mesh = plsc.VectorSubcoreMesh(core_axis_name="c", subcore_axis_name="s")
    f = functools.partial(
        pl.kernel,
        mesh=mesh,
        out_type=jax.ShapeDtypeStruct((NC, N_NODES, CH), jnp.float32),
        scratch_types=[
            pltpu.VMEM((SC_WIN,), jnp.int32),
            pltpu.VMEM((SC_WIN, CH), jnp.float32),
            pltpu.VMEM_SHARED((N_NODES, CH), jnp.float32),
        ],
        compiler_params=pltpu.CompilerParams(use_tc_tiling_on_sc=False),
    )(_scatter_body)
    return f(xrot, tgt, zeros)


def kernel(atomic_numbers, edge_distance, edge_index, source_table, target_table,
           W1, b1, g1, beta1, W2, b2, g2, beta2, W3, b3, wigner_inv):
    anf = atomic_numbers.astype(jnp.float32).reshape(N_NODES, 1)
    psrc, ptgt = _node_table_stage(anf, source_table, target_table,
                                   W1[128:192], W1[192:256], b1[None])
    g = _gather_stage(edge_index[0], edge_index[1], psrc, ptgt)
    wig3 = wigner_inv[:, :, M0_COLS].reshape(N_EDGES, NC * 3)
    x_rot = _edge_stage(edge_distance, g, wig3,
                        W1, g1, beta1, W2, b2, g2, beta2, W3, b3)
    zeros = jnp.zeros((NPT, CH), jnp.float32)
    out = _scatter_stage(x_rot, edge_index[1], zeros)
    return out.transpose(1, 0, 2)
